# baseline (speedup 1.0000x reference)
import functools

import jax
import jax.numpy as jnp
from jax import lax
from jax.experimental import pallas as pl
from jax.experimental.pallas import tpu as pltpu
from jax.experimental.pallas import tpu_sc as plsc

VOCAB = 1000000
DIM = 64
N_SAMP = 16384
N_TOK = 50
B_TOTAL = N_SAMP * N_TOK
NC, NS = 2, 16
NW = NC * NS
B_PER_W = B_TOTAL // NW
S_PER_W = N_SAMP // NW
S_CHUNK = 8
CHUNK = S_CHUNK * N_TOK
G_SIZES = (128, 128, 128, 16)
N_CHUNKS = S_PER_W // S_CHUNK
NBUF = 4

_mesh = plsc.VectorSubcoreMesh(core_axis_name="c", subcore_axis_name="s")


@functools.partial(
    pl.kernel,
    mesh=_mesh,
    compiler_params=pltpu.CompilerParams(use_tc_tiling_on_sc=False),
    out_type=jax.ShapeDtypeStruct((N_SAMP, 56, 128), jnp.float32),
    scratch_types=[
        pltpu.VMEM((B_PER_W,), jnp.int32),
        pltpu.VMEM((NBUF, CHUNK, DIM), jnp.float32),
        pltpu.SemaphoreType.DMA,
        pltpu.SemaphoreType.DMA,
        pltpu.SemaphoreType.DMA,
        pltpu.SemaphoreType.DMA,
        pltpu.SemaphoreType.DMA,
    ],
)
def _gather(idx_hbm, table_hbm, out_hbm, idx_v, rows_v, sem_g, sem_w0,
            sem_w1, sem_w2, sem_w3):
    wid = lax.axis_index("s") * NC + lax.axis_index("c")
    s_base = wid * S_PER_W
    sem_w = (sem_w0, sem_w1, sem_w2, sem_w3)

    k0 = pl.multiple_of(wid * B_PER_W, 8)
    pltpu.sync_copy(idx_hbm.at[pl.ds(k0, B_PER_W)], idx_v)

    def wb_copies(g, b):
        s0 = s_base + g * S_CHUNK
        return [
            pltpu.make_async_copy(
                rows_v.at[b, pl.ds(i * N_TOK, N_TOK)],
                out_hbm.at[s0 + i, pl.ds(0, N_TOK), pl.ds(0, DIM)],
                sem_w[b],
            )
            for i in range(S_CHUNK)
        ]

    def step(g, b):
        buf = rows_v.at[b]

        @pl.when(g >= NBUF)
        def _():
            for c in wb_copies(g - NBUF, b):
                c.wait()

        off = 0
        copies = []
        for n in G_SIZES:
            copies.append(
                pltpu.async_copy(
                    table_hbm.at[idx_v.at[pl.ds(g * CHUNK + off, n)]],
                    buf.at[pl.ds(off, n)],
                    sem_g,
                )
            )
            off += n
        for c in copies:
            c.wait()

        for c in wb_copies(g, b):
            c.start()

    def body(i, _):
        go = i * NBUF
        for b in range(NBUF):
            step(go + b, b)
        return ()

    lax.fori_loop(0, N_CHUNKS // NBUF, body, (), unroll=False)

    for b in range(NBUF):
        for c in wb_copies(N_CHUNKS - NBUF + b, b):
            c.wait()


def kernel(x, embedding):
    idx = x.reshape(B_TOTAL).astype(jnp.int32)
    out = _gather(idx, embedding)
    return out[:, :N_TOK, :DIM]


# speedup vs baseline: 1.0102x; 1.0102x over previous
import functools

import jax
import jax.numpy as jnp
from jax import lax
from jax.experimental import pallas as pl
from jax.experimental.pallas import tpu as pltpu
from jax.experimental.pallas import tpu_sc as plsc

VOCAB = 1000000
DIM = 64
N_SAMP = 16384
N_TOK = 50
B_TOTAL = N_SAMP * N_TOK
NC, NS = 2, 16
NW = NC * NS
B_PER_W = B_TOTAL // NW
S_PER_W = N_SAMP // NW
S_CHUNK = 16
CHUNK = S_CHUNK * N_TOK
G_SIZES = (128, 128, 128, 128, 128, 128, 32)
N_CHUNKS = S_PER_W // S_CHUNK
NBUF = 2

_mesh = plsc.VectorSubcoreMesh(core_axis_name="c", subcore_axis_name="s")


@functools.partial(
    pl.kernel,
    mesh=_mesh,
    compiler_params=pltpu.CompilerParams(use_tc_tiling_on_sc=False),
    out_type=jax.ShapeDtypeStruct((N_SAMP, 56, 128), jnp.float32),
    scratch_types=[
        pltpu.VMEM((B_PER_W,), jnp.int32),
        pltpu.VMEM((NBUF, CHUNK, DIM), jnp.float32),
        pltpu.SemaphoreType.DMA,
        pltpu.SemaphoreType.DMA,
        pltpu.SemaphoreType.DMA,
    ],
)
def _gather(idx_hbm, table_hbm, out_hbm, idx_v, rows_v, sem_g, sem_w0,
            sem_w1):
    wid = lax.axis_index("s") * NC + lax.axis_index("c")
    s_base = wid * S_PER_W
    sem_w = (sem_w0, sem_w1)

    k0 = pl.multiple_of(wid * B_PER_W, 8)
    pltpu.sync_copy(idx_hbm.at[pl.ds(k0, B_PER_W)], idx_v)

    def wb_copies(g, b):
        s0 = s_base + g * S_CHUNK
        return [
            pltpu.make_async_copy(
                rows_v.at[b, pl.ds(i * N_TOK, N_TOK)],
                out_hbm.at[s0 + i, pl.ds(0, N_TOK), pl.ds(0, DIM)],
                sem_w[b],
            )
            for i in range(S_CHUNK)
        ]

    def step(g, b):
        buf = rows_v.at[b]

        @pl.when(g >= NBUF)
        def _():
            for c in wb_copies(g - NBUF, b):
                c.wait()

        off = 0
        copies = []
        for n in G_SIZES:
            copies.append(
                pltpu.async_copy(
                    table_hbm.at[idx_v.at[pl.ds(g * CHUNK + off, n)]],
                    buf.at[pl.ds(off, n)],
                    sem_g,
                )
            )
            off += n
        for c in copies:
            c.wait()

        for c in wb_copies(g, b):
            c.start()

    def body(i, _):
        go = i * NBUF
        for b in range(NBUF):
            step(go + b, b)
        return ()

    lax.fori_loop(0, N_CHUNKS // NBUF, body, (), unroll=False)

    for b in range(NBUF):
        for c in wb_copies(N_CHUNKS - NBUF + b, b):
            c.wait()


def kernel(x, embedding):
    idx = x.reshape(B_TOTAL).astype(jnp.int32)
    out = _gather(idx, embedding)
    return out[:, :N_TOK, :DIM]
